# baseline (speedup 1.0000x reference)
import functools

import jax
import jax.numpy as jnp
from jax import lax
from jax.experimental import pallas as pl
from jax.experimental.pallas import tpu as pltpu

B = 1024
INPUT_DIM = 256
M = 65536
K = 64
MULTI_K = 256
AUX_K = 128

BM = 2048


def _matmul_body(x_ref, w_ref, ib_ref, nb_ref, out_ref, zeros_ref):
    xc = x_ref[...] - ib_ref[...]
    acc = lax.dot_general(
        xc, w_ref[...],
        dimension_numbers=(((1,), (1,)), ((), ())),
        preferred_element_type=jnp.float32,
    )
    out_ref[...] = acc + nb_ref[...]
    zeros_ref[...] = jnp.zeros_like(zeros_ref)


def _encoder_matmul(x, W_enc, input_bias, neuron_bias):
    grid = (M // BM,)
    return pl.pallas_call(
        _matmul_body,
        grid=grid,
        in_specs=[
            pl.BlockSpec((B, INPUT_DIM), lambda i: (0, 0)),
            pl.BlockSpec((BM, INPUT_DIM), lambda i: (i, 0)),
            pl.BlockSpec((1, INPUT_DIM), lambda i: (0, 0)),
            pl.BlockSpec((1, BM), lambda i: (0, i)),
        ],
        out_specs=[
            pl.BlockSpec((B, BM), lambda i: (0, i)),
            pl.BlockSpec((B, BM), lambda i: (0, i)),
        ],
        out_shape=[
            jax.ShapeDtypeStruct((B, M), jnp.float32),
            jax.ShapeDtypeStruct((B, M), jnp.float32),
        ],
    )(x, W_enc, input_bias.reshape(1, INPUT_DIM), neuron_bias.reshape(1, M))


def kernel(x, W_enc, W_dec, input_bias, neuron_bias):
    pre_act, act_zeros = _encoder_matmul(x, W_enc, input_bias, neuron_bias)

    mk_vals, mk_idx = jax.lax.top_k(pre_act, MULTI_K)
    mk_vals_relu = jax.nn.relu(mk_vals)
    topk_indices = mk_idx[:, :K]
    topk_values = mk_vals_relu[:, :K]

    rows = jnp.arange(B)[:, None]
    activations = act_zeros.at[rows, topk_indices].set(topk_values)

    W_dec_T = W_dec.T
    gathered = W_dec_T[mk_idx.reshape(-1)].reshape(B, MULTI_K, INPUT_DIM)
    reconstruction = (
        jnp.einsum("bk,bkd->bd", mk_vals_relu[:, :K], gathered[:, :K]) + input_bias
    )
    multik_reconstruction = (
        jnp.einsum("bk,bkd->bd", mk_vals_relu, gathered) + input_bias
    )

    aux_values = jnp.zeros((B, AUX_K), dtype=jnp.float32)
    pos = jnp.arange(M, dtype=jnp.int32)
    score = jnp.where(jnp.signbit(pre_act), -(pos + M), -pos)
    aux_score, _ = jax.lax.top_k(score, AUX_K)
    aux_indices = jnp.where(aux_score <= -M, -aux_score - M, -aux_score)

    return (reconstruction, activations, topk_indices, topk_values,
            multik_reconstruction, aux_indices, aux_values)


# speedup vs baseline: 8.2915x; 8.2915x over previous
import functools

import jax
import jax.numpy as jnp
from jax import lax
from jax.experimental import pallas as pl
from jax.experimental.pallas import tpu as pltpu
from jax.experimental.pallas import tpu_sc as plsc

B = 1024
INPUT_DIM = 256
M = 65536
K = 64
MULTI_K = 256
AUX_K = 128

BM = 2048

NC = 2
NS = 16
L = 16
NW = NC * NS
ROWS_PER_W = B // NW
NV = M // L
NACC = 32
CAP = 12288
CANDS = 512
AUXBUF = 144


def _matmul_body(x_ref, w_ref, ib_ref, nb_ref, out_ref, zeros_ref):
    xc = x_ref[...] - ib_ref[...]
    acc = lax.dot_general(
        xc, w_ref[...],
        dimension_numbers=(((1,), (1,)), ((), ())),
        preferred_element_type=jnp.float32,
    )
    out_ref[...] = acc + nb_ref[...]
    zeros_ref[...] = jnp.zeros_like(zeros_ref)


def _encoder_matmul(x, W_enc, input_bias, neuron_bias):
    return pl.pallas_call(
        _matmul_body,
        grid=(M // BM,),
        in_specs=[
            pl.BlockSpec((B, INPUT_DIM), lambda i: (0, 0)),
            pl.BlockSpec((BM, INPUT_DIM), lambda i: (i, 0)),
            pl.BlockSpec((1, INPUT_DIM), lambda i: (0, 0)),
            pl.BlockSpec((1, BM), lambda i: (0, i)),
        ],
        out_specs=[
            pl.BlockSpec((B, BM), lambda i: (0, i)),
            pl.BlockSpec((B, BM), lambda i: (0, i)),
        ],
        out_shape=[
            jax.ShapeDtypeStruct((B, M), jnp.float32),
            jax.ShapeDtypeStruct((B, M), jnp.float32),
        ],
    )(x, W_enc, input_bias.reshape(1, INPUT_DIM), neuron_bias.reshape(1, M))


def _to_u32(x):
    b = plsc.bitcast(x, jnp.int32)
    m = lax.shift_right_arithmetic(b, 31)
    u = b ^ (m | jnp.int32(-2147483648))
    return plsc.bitcast(u, jnp.uint32)


def _from_u32(u):
    s = plsc.bitcast(u, jnp.int32)
    mask = jnp.where(s < 0, jnp.int32(-2147483648), jnp.int32(-1))
    return plsc.bitcast(s ^ mask, jnp.float32)


def _sc_topk_body(pre_ref, cvals_ref, cidx_ref, aux_ref,
                  data, gmaxu, cand, candu, hi_idx, eq_idx, ovals, oidx, auxb):
    wid = lax.axis_index("s") * NC + lax.axis_index("c")
    iota = lax.iota(jnp.int32, L)
    ninf = jnp.full((L,), -jnp.inf, jnp.float32)
    zeros_i = jnp.zeros((L,), jnp.int32)

    def count_ge(ref, nvecs, midv):
        def cb(k, acc):
            g = ref[pl.ds(k * L, L)]
            return acc + (g >= midv).astype(jnp.int32)
        accv = lax.fori_loop(0, nvecs, cb, zeros_i)
        return plsc.cumsum(accv)[L - 1]

    def bsearch(ref, nvecs, iters, target):
        def bs(_, lohi):
            lo, hi = lohi
            mid = lo + lax.shift_right_logical(hi - lo, jnp.uint32(1))
            midv = jnp.full((L,), mid, jnp.uint32)
            c = count_ge(ref, nvecs, midv)
            take = c >= target
            return (jnp.where(take, mid, lo), jnp.where(take, hi, mid))
        lo, _ = lax.fori_loop(
            0, iters, bs, (jnp.uint32(0), jnp.uint32(0xFFFFFFFF)))
        return lo

    def append(dst, off_scalar, idxv, msk, cap):
        ranks = plsc.cumsum(msk.astype(jnp.int32))
        offv = jnp.full((L,), off_scalar, jnp.int32)
        pos = offv + ranks - 1
        m2 = msk & (pos < cap)
        plsc.store_scatter(dst, [pos], idxv, mask=m2)
        pc = plsc.all_reduce_population_count(msk)
        return jnp.minimum(off_scalar + pc[0], cap)

    def per_row(j, _):
        row = wid * ROWS_PER_W + j
        pltpu.sync_copy(pre_ref.at[row], data)

        def aux_scan(want_nonneg):
            def cond(c):
                i, off = c
                return (off < AUX_K) & (i < NV)
            def body(c):
                i, off = c
                v = data[pl.ds(i * L, L)]
                bits = plsc.bitcast(v, jnp.int32)
                msk = (bits >= 0) if want_nonneg else (bits < 0)
                posidx = jnp.full((L,), i * L, jnp.int32) + iota
                off = append(auxb, off, posidx, msk, AUXBUF)
                return (i + 1, off)
            return body, cond
        b1, c1 = aux_scan(True)
        _, aoff = lax.while_loop(c1, b1, (jnp.int32(0), jnp.int32(0)))
        b2, c2 = aux_scan(False)
        lax.while_loop(c2, b2, (jnp.int32(0), aoff))

        def pa(i, carry):
            base = i * (NACC * L)
            return tuple(
                jnp.maximum(carry[t], data[pl.ds(base + t * L, L)])
                for t in range(NACC))
        maxes = lax.fori_loop(0, NV // NACC, pa, (ninf,) * NACC)
        for t in range(NACC):
            gmaxu[pl.ds(t * L, L)] = _to_u32(maxes[t])

        t1 = bsearch(gmaxu, NACC, 18, MULTI_K)
        thrv = _from_u32(jnp.full((L,), t1, jnp.uint32))

        def pb(i, carry):
            offv, idxv = carry
            v = data[pl.ds(i * L, L)]
            msk = v >= thrv
            ranks = plsc.cumsum(msk.astype(jnp.int32))
            pos = offv + ranks - 1
            m2 = msk & (pos < CAP)
            plsc.store_scatter(cand, [pos], idxv, mask=m2)
            pc = plsc.all_reduce_population_count(msk)
            return (jnp.minimum(offv + pc, CAP), idxv + L)
        offv, _ = lax.fori_loop(0, NV, pb, (zeros_i, iota), unroll=8)
        count = offv[0]
        countv = jnp.full((L,), count, jnp.int32)

        def emit_main():
            for k in range(CANDS // L):
                lanepos = jnp.full((L,), k * L, jnp.int32) + iota
                valid = lanepos < countv
                idxv = cand[pl.ds(k * L, L)]
                idxs = jnp.where(valid, idxv, 0)
                vals = plsc.load_gather(data, [idxs])
                ovals[pl.ds(k * L, L)] = jnp.where(valid, vals, ninf)
                oidx[pl.ds(k * L, L)] = idxs

        def hi_eq_compact_emit(src_u, src_idx, nvecs, tstar, nv_is_row):
            tsv = jnp.full((L,), tstar, jnp.uint32)
            def cc(k, carry):
                hoff, eoff = carry
                u = src_u(k)
                idxv = src_idx(k)
                mh = u > tsv
                me = u == tsv
                hoff = append(hi_idx, hoff, idxv, mh, MULTI_K + L)
                eoff = append(eq_idx, eoff, idxv, me, MULTI_K)
                return (hoff, eoff)
            hoff, eoff = lax.fori_loop(
                0, nvecs, cc, (jnp.int32(0), jnp.int32(0)))
            hv = jnp.full((L,), hoff, jnp.int32)
            ev = jnp.full((L,), eoff, jnp.int32)
            for k in range(MULTI_K // L):
                lanepos = jnp.full((L,), k * L, jnp.int32) + iota
                validh = lanepos < hv
                valide = lanepos < ev
                ih = jnp.where(validh, hi_idx[pl.ds(k * L, L)], 0)
                ie = jnp.where(valide, eq_idx[pl.ds(k * L, L)], 0)
                if nv_is_row:
                    vh = _from_u32(plsc.bitcast(
                        plsc.load_gather(data, [ih]), jnp.uint32))
                    ve = _from_u32(plsc.bitcast(
                        plsc.load_gather(data, [ie]), jnp.uint32))
                else:
                    vh = plsc.load_gather(data, [ih])
                    ve = plsc.load_gather(data, [ie])
                ovals[pl.ds(k * L, L)] = jnp.where(validh, vh, ninf)
                oidx[pl.ds(k * L, L)] = ih
                ovals[pl.ds(MULTI_K + k * L, L)] = jnp.where(valide, ve, ninf)
                oidx[pl.ds(MULTI_K + k * L, L)] = ie

        def emit_refine():
            nk = lax.shift_right_logical(count + (L - 1), jnp.int32(4))
            def uf(k, _):
                lanepos = jnp.full((L,), 0, jnp.int32) + iota + k * L
                valid = lanepos < countv
                idxv = cand[pl.ds(k * L, L)]
                idxs = jnp.where(valid, idxv, 0)
                vals = plsc.load_gather(data, [idxs])
                u = _to_u32(vals)
                candu[pl.ds(k * L, L)] = jnp.where(valid, u, jnp.uint32(0))
                return 0
            lax.fori_loop(0, nk, uf, 0)
            tstar = bsearch(candu, nk, 32, MULTI_K)
            hi_eq_compact_emit(
                lambda k: candu[pl.ds(k * L, L)],
                lambda k: cand[pl.ds(k * L, L)],
                nk, tstar, False)

        def emit_overflow():
            def ip(i, _):
                v = data[pl.ds(i * L, L)]
                data[pl.ds(i * L, L)] = plsc.bitcast(
                    plsc.bitcast(_to_u32(v), jnp.int32), jnp.float32)
                return 0
            lax.fori_loop(0, NV, ip, 0)
            def src_u(k):
                return plsc.bitcast(data[pl.ds(k * L, L)], jnp.uint32)
            def count_ge_row(midv):
                def cb(k, acc):
                    return acc + (src_u(k) >= midv).astype(jnp.int32)
                return jnp.sum(lax.fori_loop(0, NV, cb, zeros_i))
            def bs(_, lohi):
                lo, hi = lohi
                mid = lo + lax.shift_right_logical(hi - lo, jnp.uint32(1))
                c = count_ge_row(jnp.full((L,), mid, jnp.uint32))
                take = c >= MULTI_K
                return (jnp.where(take, mid, lo), jnp.where(take, hi, mid))
            tstar, _ = lax.fori_loop(
                0, 32, bs, (jnp.uint32(0), jnp.uint32(0xFFFFFFFF)))
            def src_idx(k):
                return jnp.full((L,), k * L, jnp.int32) + iota
            hi_eq_compact_emit(src_u, src_idx, NV, tstar, True)

        lax.cond(
            count <= CANDS, emit_main,
            lambda: lax.cond(count < CAP, emit_refine, emit_overflow))

        pltpu.sync_copy(ovals, cvals_ref.at[row])
        pltpu.sync_copy(oidx, cidx_ref.at[row])
        pltpu.sync_copy(auxb.at[pl.ds(0, AUX_K)], aux_ref.at[row])
        return 0

    lax.fori_loop(0, ROWS_PER_W, per_row, 0)


def _sc_topk(pre_act):
    fn = pl.kernel(
        _sc_topk_body,
        compiler_params=pltpu.CompilerParams(needs_layout_passes=False),
        out_type=[
            jax.ShapeDtypeStruct((B, CANDS), jnp.float32),
            jax.ShapeDtypeStruct((B, CANDS), jnp.int32),
            jax.ShapeDtypeStruct((B, AUX_K), jnp.int32),
        ],
        mesh=plsc.VectorSubcoreMesh(
            core_axis_name="c", subcore_axis_name="s", num_cores=NC),
        scratch_types=[
            pltpu.VMEM((M,), jnp.float32),
            pltpu.VMEM((NACC * L,), jnp.uint32),
            pltpu.VMEM((CAP + L,), jnp.int32),
            pltpu.VMEM((CAP + L,), jnp.uint32),
            pltpu.VMEM((MULTI_K + 2 * L,), jnp.int32),
            pltpu.VMEM((MULTI_K + L,), jnp.int32),
            pltpu.VMEM((CANDS,), jnp.float32),
            pltpu.VMEM((CANDS,), jnp.int32),
            pltpu.VMEM((AUXBUF,), jnp.int32),
        ],
    )
    return fn(pre_act)


def kernel(x, W_enc, W_dec, input_bias, neuron_bias):
    pre_act, act_zeros = _encoder_matmul(x, W_enc, input_bias, neuron_bias)

    cvals, cidx, aux_indices = _sc_topk(pre_act)

    mk_vals, mk_pos = jax.lax.top_k(cvals, MULTI_K)
    mk_idx = jnp.take_along_axis(cidx, mk_pos, axis=1)
    mk_vals_relu = jax.nn.relu(mk_vals)
    topk_indices = mk_idx[:, :K]
    topk_values = mk_vals_relu[:, :K]

    rows = jnp.arange(B)[:, None]
    activations = act_zeros.at[rows, topk_indices].set(topk_values)

    W_dec_T = W_dec.T
    gathered = W_dec_T[mk_idx.reshape(-1)].reshape(B, MULTI_K, INPUT_DIM)
    reconstruction = (
        jnp.einsum("bk,bkd->bd", mk_vals_relu[:, :K], gathered[:, :K]) + input_bias
    )
    multik_reconstruction = (
        jnp.einsum("bk,bkd->bd", mk_vals_relu, gathered) + input_bias
    )

    aux_values = jnp.zeros((B, AUX_K), dtype=jnp.float32)
    return (reconstruction, activations, topk_indices, topk_values,
            multik_reconstruction, aux_indices, aux_values)


# speedup vs baseline: 27.6518x; 3.3349x over previous
import functools

import jax
import jax.numpy as jnp
from jax import lax
from jax.experimental import pallas as pl
from jax.experimental.pallas import tpu as pltpu
from jax.experimental.pallas import tpu_sc as plsc

B = 1024
INPUT_DIM = 256
M = 65536
K = 64
MULTI_K = 256
AUX_K = 128

BM = 2048

NC = 2
NS = 16
L = 16
NW = NC * NS
ROWS_PER_W = B // NW
NV = M // L
NACC = 32
CAP = 12288
CANDS = 512
AUXBUF = 144


def _matmul_body(x_ref, w_ref, ib_ref, nb_ref, out_ref, zeros_ref):
    xc = x_ref[...] - ib_ref[...]
    acc = lax.dot_general(
        xc, w_ref[...],
        dimension_numbers=(((1,), (1,)), ((), ())),
        preferred_element_type=jnp.float32,
    )
    out_ref[...] = acc + nb_ref[...]
    zeros_ref[...] = jnp.zeros_like(zeros_ref)


def _encoder_matmul(x, W_enc, input_bias, neuron_bias):
    return pl.pallas_call(
        _matmul_body,
        grid=(M // BM,),
        in_specs=[
            pl.BlockSpec((B, INPUT_DIM), lambda i: (0, 0)),
            pl.BlockSpec((BM, INPUT_DIM), lambda i: (i, 0)),
            pl.BlockSpec((1, INPUT_DIM), lambda i: (0, 0)),
            pl.BlockSpec((1, BM), lambda i: (0, i)),
        ],
        out_specs=[
            pl.BlockSpec((B, BM), lambda i: (0, i)),
            pl.BlockSpec((B, BM), lambda i: (0, i)),
        ],
        out_shape=[
            jax.ShapeDtypeStruct((B, M), jnp.float32),
            jax.ShapeDtypeStruct((B, M), jnp.float32),
        ],
    )(x, W_enc, input_bias.reshape(1, INPUT_DIM), neuron_bias.reshape(1, M))


def _to_u32(x):
    b = plsc.bitcast(x, jnp.int32)
    m = lax.shift_right_arithmetic(b, 31)
    u = b ^ (m | jnp.int32(-2147483648))
    return plsc.bitcast(u, jnp.uint32)


def _from_u32(u):
    s = plsc.bitcast(u, jnp.int32)
    mask = jnp.where(s < 0, jnp.int32(-2147483648), jnp.int32(-1))
    return plsc.bitcast(s ^ mask, jnp.float32)


def _sc_topk_body(pre_ref, cvals_ref, cidx_ref, aux_ref,
                  data, gmaxu, cand, candu, hi_idx, eq_idx, ovals, oidx, auxb):
    wid = lax.axis_index("s") * NC + lax.axis_index("c")
    iota = lax.iota(jnp.int32, L)
    ninf = jnp.full((L,), -jnp.inf, jnp.float32)
    zeros_i = jnp.zeros((L,), jnp.int32)

    def count_ge(ref, nvecs, midv):
        def cb(k, acc):
            g = ref[pl.ds(k * L, L)]
            return acc + (g >= midv).astype(jnp.int32)
        accv = lax.fori_loop(0, nvecs, cb, zeros_i)
        return plsc.cumsum(accv)[L - 1]

    def bsearch(ref, nvecs, iters, target):
        def bs(_, lohi):
            lo, hi = lohi
            mid = lo + lax.shift_right_logical(hi - lo, jnp.uint32(1))
            midv = jnp.full((L,), mid, jnp.uint32)
            c = count_ge(ref, nvecs, midv)
            take = c >= target
            return (jnp.where(take, mid, lo), jnp.where(take, hi, mid))
        lo, _ = lax.fori_loop(
            0, iters, bs, (jnp.uint32(0), jnp.uint32(0xFFFFFFFF)))
        return lo

    def append(dst, off_scalar, idxv, msk, cap):
        plsc.store_compressed(dst.at[pl.ds(off_scalar, L)], idxv, mask=msk)
        pc = plsc.all_reduce_population_count(msk)
        return jnp.minimum(off_scalar + pc[0], cap)

    def per_row(j, _):
        row = wid * ROWS_PER_W + j
        pltpu.sync_copy(pre_ref.at[row], data)

        def aux_scan(want_nonneg):
            def cond(c):
                i, off = c
                return (off < AUX_K) & (i < NV)
            def body(c):
                i, off = c
                v = data[pl.ds(i * L, L)]
                bits = plsc.bitcast(v, jnp.int32)
                msk = (bits >= 0) if want_nonneg else (bits < 0)
                posidx = jnp.full((L,), i * L, jnp.int32) + iota
                off = append(auxb, off, posidx, msk, AUX_K)
                return (i + 1, off)
            return body, cond
        b1, c1 = aux_scan(True)
        _, aoff = lax.while_loop(c1, b1, (jnp.int32(0), jnp.int32(0)))
        b2, c2 = aux_scan(False)
        lax.while_loop(c2, b2, (jnp.int32(0), aoff))

        def pa(i, carry):
            base = i * (NACC * L)
            return tuple(
                jnp.maximum(carry[t], data[pl.ds(base + t * L, L)])
                for t in range(NACC))
        maxes = lax.fori_loop(0, NV // NACC, pa, (ninf,) * NACC)
        for t in range(NACC):
            gmaxu[pl.ds(t * L, L)] = _to_u32(maxes[t])

        t1 = bsearch(gmaxu, NACC, 18, MULTI_K)
        thrv = _from_u32(jnp.full((L,), t1, jnp.uint32))

        def pb(i, carry):
            off, idxv = carry
            v = data[pl.ds(i * L, L)]
            msk = v >= thrv
            plsc.store_compressed(cand.at[pl.ds(off, L)], idxv, mask=msk)
            pc = plsc.all_reduce_population_count(msk)
            return (jnp.minimum(off + pc[0], CAP), idxv + L)
        count, _ = lax.fori_loop(
            0, NV, pb, (jnp.int32(0), iota), unroll=8)
        countv = jnp.full((L,), count, jnp.int32)

        def emit_main():
            for k in range(CANDS // L):
                lanepos = jnp.full((L,), k * L, jnp.int32) + iota
                valid = lanepos < countv
                idxv = cand[pl.ds(k * L, L)]
                idxs = jnp.where(valid, idxv, 0)
                vals = plsc.load_gather(data, [idxs])
                ovals[pl.ds(k * L, L)] = jnp.where(valid, vals, ninf)
                oidx[pl.ds(k * L, L)] = idxs

        def hi_eq_compact_emit(src_u, src_idx, nvecs, tstar, nv_is_row):
            tsv = jnp.full((L,), tstar, jnp.uint32)
            def cc(k, carry):
                hoff, eoff = carry
                u = src_u(k)
                idxv = src_idx(k)
                mh = u > tsv
                me = u == tsv
                hoff = append(hi_idx, hoff, idxv, mh, MULTI_K + L)
                eoff = append(eq_idx, eoff, idxv, me, MULTI_K)
                return (hoff, eoff)
            hoff, eoff = lax.fori_loop(
                0, nvecs, cc, (jnp.int32(0), jnp.int32(0)))
            hv = jnp.full((L,), hoff, jnp.int32)
            ev = jnp.full((L,), eoff, jnp.int32)
            for k in range(MULTI_K // L):
                lanepos = jnp.full((L,), k * L, jnp.int32) + iota
                validh = lanepos < hv
                valide = lanepos < ev
                ih = jnp.where(validh, hi_idx[pl.ds(k * L, L)], 0)
                ie = jnp.where(valide, eq_idx[pl.ds(k * L, L)], 0)
                if nv_is_row:
                    vh = _from_u32(plsc.bitcast(
                        plsc.load_gather(data, [ih]), jnp.uint32))
                    ve = _from_u32(plsc.bitcast(
                        plsc.load_gather(data, [ie]), jnp.uint32))
                else:
                    vh = plsc.load_gather(data, [ih])
                    ve = plsc.load_gather(data, [ie])
                ovals[pl.ds(k * L, L)] = jnp.where(validh, vh, ninf)
                oidx[pl.ds(k * L, L)] = ih
                ovals[pl.ds(MULTI_K + k * L, L)] = jnp.where(valide, ve, ninf)
                oidx[pl.ds(MULTI_K + k * L, L)] = ie

        def emit_refine():
            nk = lax.shift_right_logical(count + (L - 1), jnp.int32(4))
            def uf(k, _):
                lanepos = jnp.full((L,), 0, jnp.int32) + iota + k * L
                valid = lanepos < countv
                idxv = cand[pl.ds(k * L, L)]
                idxs = jnp.where(valid, idxv, 0)
                vals = plsc.load_gather(data, [idxs])
                u = _to_u32(vals)
                candu[pl.ds(k * L, L)] = jnp.where(valid, u, jnp.uint32(0))
                return 0
            lax.fori_loop(0, nk, uf, 0)
            tstar = bsearch(candu, nk, 32, MULTI_K)
            hi_eq_compact_emit(
                lambda k: candu[pl.ds(k * L, L)],
                lambda k: cand[pl.ds(k * L, L)],
                nk, tstar, False)

        def emit_overflow():
            def ip(i, _):
                v = data[pl.ds(i * L, L)]
                data[pl.ds(i * L, L)] = plsc.bitcast(
                    plsc.bitcast(_to_u32(v), jnp.int32), jnp.float32)
                return 0
            lax.fori_loop(0, NV, ip, 0)
            def src_u(k):
                return plsc.bitcast(data[pl.ds(k * L, L)], jnp.uint32)
            def count_ge_row(midv):
                def cb(k, acc):
                    return acc + (src_u(k) >= midv).astype(jnp.int32)
                return jnp.sum(lax.fori_loop(0, NV, cb, zeros_i))
            def bs(_, lohi):
                lo, hi = lohi
                mid = lo + lax.shift_right_logical(hi - lo, jnp.uint32(1))
                c = count_ge_row(jnp.full((L,), mid, jnp.uint32))
                take = c >= MULTI_K
                return (jnp.where(take, mid, lo), jnp.where(take, hi, mid))
            tstar, _ = lax.fori_loop(
                0, 32, bs, (jnp.uint32(0), jnp.uint32(0xFFFFFFFF)))
            def src_idx(k):
                return jnp.full((L,), k * L, jnp.int32) + iota
            hi_eq_compact_emit(src_u, src_idx, NV, tstar, True)

        lax.cond(
            count <= CANDS, emit_main,
            lambda: lax.cond(count < CAP, emit_refine, emit_overflow))

        pltpu.sync_copy(ovals, cvals_ref.at[row])
        pltpu.sync_copy(oidx, cidx_ref.at[row])
        pltpu.sync_copy(auxb.at[pl.ds(0, AUX_K)], aux_ref.at[row])
        return 0

    lax.fori_loop(0, ROWS_PER_W, per_row, 0)


def _sc_topk(pre_act):
    fn = pl.kernel(
        _sc_topk_body,
        compiler_params=pltpu.CompilerParams(needs_layout_passes=False),
        out_type=[
            jax.ShapeDtypeStruct((B, CANDS), jnp.float32),
            jax.ShapeDtypeStruct((B, CANDS), jnp.int32),
            jax.ShapeDtypeStruct((B, AUX_K), jnp.int32),
        ],
        mesh=plsc.VectorSubcoreMesh(
            core_axis_name="c", subcore_axis_name="s", num_cores=NC),
        scratch_types=[
            pltpu.VMEM((M,), jnp.float32),
            pltpu.VMEM((NACC * L,), jnp.uint32),
            pltpu.VMEM((CAP + L,), jnp.int32),
            pltpu.VMEM((CAP + L,), jnp.uint32),
            pltpu.VMEM((MULTI_K + 2 * L,), jnp.int32),
            pltpu.VMEM((MULTI_K + L,), jnp.int32),
            pltpu.VMEM((CANDS,), jnp.float32),
            pltpu.VMEM((CANDS,), jnp.int32),
            pltpu.VMEM((AUXBUF,), jnp.int32),
        ],
    )
    return fn(pre_act)


SCH = B * K // 128


def _sc_scatter_body(idx_hbm, val_hbm, flat_ref, idxv, valv, sem):
    wid = lax.axis_index("s") * NC + lax.axis_index("c")
    chunks = SCH // NW
    base_c = wid * chunks
    iota = lax.iota(jnp.int32, L)
    pltpu.sync_copy(idx_hbm.at[pl.ds(base_c, chunks)], idxv)
    pltpu.sync_copy(val_hbm.at[pl.ds(base_c, chunks)], valv)
    for c in range(chunks):
        for q in range(128 // L):
            pos = jnp.full((L,), (base_c + c) * 128 + q * L, jnp.int32) + iota
            rowv = lax.shift_right_logical(pos, 6)
            basev = lax.shift_left(rowv, 16)
            idxv[c, pl.ds(q * L, L)] = idxv[c, pl.ds(q * L, L)] + basev
    for c in range(chunks):
        pltpu.async_copy(valv.at[c], flat_ref.at[idxv.at[c]], sem).wait()


def _sc_scatter(idx_pad, val_pad, flat_ref):
    fn = pl.kernel(
        _sc_scatter_body,
        compiler_params=pltpu.CompilerParams(needs_layout_passes=False),
        out_type=(),
        mesh=plsc.VectorSubcoreMesh(
            core_axis_name="c", subcore_axis_name="s", num_cores=NC),
        scratch_types=[
            pltpu.VMEM((SCH // NW, 128), jnp.int32),
            pltpu.VMEM((SCH // NW, 128), jnp.float32),
            pltpu.SemaphoreType.DMA,
        ],
    )
    fn(idx_pad, val_pad, flat_ref)


def _sc_recon_body(wdt_hbm, idx_hbm, val_hbm, bias_hbm, r_hbm, mr_hbm,
                   idxv, valv, gat, bias, rbuf, mrbuf, sem):
    wid = lax.axis_index("s") * NC + lax.axis_index("c")
    pltpu.sync_copy(bias_hbm, bias)
    nd = INPUT_DIM // L

    def per_row(j, _):
        row = wid * ROWS_PER_W + j
        pltpu.sync_copy(idx_hbm.at[row], idxv)
        pltpu.sync_copy(val_hbm.at[row], valv)
        pltpu.async_copy(wdt_hbm.at[idxv], gat, sem).wait()
        accs = tuple(bias[pl.ds(d * L, L)] for d in range(nd))

        def macbody(t, accs):
            sv = plsc.load_gather(valv, [jnp.full((L,), t, jnp.int32)])
            return tuple(
                a + sv * gat[t, pl.ds(d * L, L)] for d, a in enumerate(accs))

        accs = lax.fori_loop(0, K, macbody, accs, unroll=4)
        for d in range(nd):
            rbuf[pl.ds(d * L, L)] = accs[d]
        accs = lax.fori_loop(K, MULTI_K, macbody, accs, unroll=4)
        for d in range(nd):
            mrbuf[pl.ds(d * L, L)] = accs[d]
        pltpu.sync_copy(rbuf, r_hbm.at[row])
        pltpu.sync_copy(mrbuf, mr_hbm.at[row])
        return 0

    lax.fori_loop(0, ROWS_PER_W, per_row, 0)


def _sc_recon(W_dec_T, mk_idx, mk_vals_relu, input_bias):
    fn = pl.kernel(
        _sc_recon_body,
        compiler_params=pltpu.CompilerParams(needs_layout_passes=False),
        out_type=[
            jax.ShapeDtypeStruct((B, INPUT_DIM), jnp.float32),
            jax.ShapeDtypeStruct((B, INPUT_DIM), jnp.float32),
        ],
        mesh=plsc.VectorSubcoreMesh(
            core_axis_name="c", subcore_axis_name="s", num_cores=NC),
        scratch_types=[
            pltpu.VMEM((MULTI_K,), jnp.int32),
            pltpu.VMEM((MULTI_K,), jnp.float32),
            pltpu.VMEM((MULTI_K, INPUT_DIM), jnp.float32),
            pltpu.VMEM((INPUT_DIM,), jnp.float32),
            pltpu.VMEM((INPUT_DIM,), jnp.float32),
            pltpu.VMEM((INPUT_DIM,), jnp.float32),
            pltpu.SemaphoreType.DMA,
        ],
    )
    return fn(W_dec_T, mk_idx, mk_vals_relu, input_bias)


def kernel(x, W_enc, W_dec, input_bias, neuron_bias):
    pre_act, act_zeros = _encoder_matmul(x, W_enc, input_bias, neuron_bias)

    cvals, cidx, aux_indices = _sc_topk(pre_act)

    mk_vals, mk_pos = jax.lax.top_k(cvals, MULTI_K)
    mk_idx = jnp.take_along_axis(cidx, mk_pos, axis=1)
    mk_vals_relu = jax.nn.relu(mk_vals)
    topk_indices = mk_idx[:, :K]
    topk_values = mk_vals_relu[:, :K]

    flat_ref = jax.new_ref(act_zeros.reshape(B * M))
    _sc_scatter(topk_indices.reshape(SCH, 128),
                topk_values.reshape(SCH, 128), flat_ref)
    activations = flat_ref[...].reshape(B, M)

    W_dec_T = W_dec.T
    reconstruction, multik_reconstruction = _sc_recon(
        W_dec_T, mk_idx, mk_vals_relu, input_bias)

    aux_values = jnp.zeros((B, AUX_K), dtype=jnp.float32)
    return (reconstruction, activations, topk_indices, topk_values,
            multik_reconstruction, aux_indices, aux_values)


# speedup vs baseline: 28.8205x; 1.0423x over previous
import functools

import jax
import jax.numpy as jnp
from jax import lax
from jax.experimental import pallas as pl
from jax.experimental.pallas import tpu as pltpu
from jax.experimental.pallas import tpu_sc as plsc

B = 1024
INPUT_DIM = 256
M = 65536
K = 64
MULTI_K = 256
AUX_K = 128

BM = 2048

NC = 2
NS = 16
L = 16
NW = NC * NS
ROWS_PER_W = B // NW
NV = M // L
NACC = 32
CAP = 12288
CANDS = 512
AUXBUF = 144


def _matmul_body(x_ref, w_ref, wd_ref, ib_ref, nb_ref,
                 out_ref, zeros_ref, wdt_ref):
    xc = x_ref[...] - ib_ref[...]
    acc = lax.dot_general(
        xc, w_ref[...],
        dimension_numbers=(((1,), (1,)), ((), ())),
        preferred_element_type=jnp.float32,
    )
    out_ref[...] = acc + nb_ref[...]
    zeros_ref[...] = jnp.zeros_like(zeros_ref)
    wdt_ref[...] = wd_ref[...].T


def _encoder_matmul(x, W_enc, W_dec, input_bias, neuron_bias):
    return pl.pallas_call(
        _matmul_body,
        grid=(M // BM,),
        in_specs=[
            pl.BlockSpec((B, INPUT_DIM), lambda i: (0, 0)),
            pl.BlockSpec((BM, INPUT_DIM), lambda i: (i, 0)),
            pl.BlockSpec((INPUT_DIM, BM), lambda i: (0, i)),
            pl.BlockSpec((1, INPUT_DIM), lambda i: (0, 0)),
            pl.BlockSpec((1, BM), lambda i: (0, i)),
        ],
        out_specs=[
            pl.BlockSpec((B, BM), lambda i: (0, i)),
            pl.BlockSpec((B, BM), lambda i: (0, i)),
            pl.BlockSpec((BM, INPUT_DIM), lambda i: (i, 0)),
        ],
        out_shape=[
            jax.ShapeDtypeStruct((B, M), jnp.float32),
            jax.ShapeDtypeStruct((B, M), jnp.float32),
            jax.ShapeDtypeStruct((M, INPUT_DIM), jnp.float32),
        ],
    )(x, W_enc, W_dec, input_bias.reshape(1, INPUT_DIM),
      neuron_bias.reshape(1, M))


def _to_u32(x):
    b = plsc.bitcast(x, jnp.int32)
    m = lax.shift_right_arithmetic(b, 31)
    u = b ^ (m | jnp.int32(-2147483648))
    return plsc.bitcast(u, jnp.uint32)


def _from_u32(u):
    s = plsc.bitcast(u, jnp.int32)
    mask = jnp.where(s < 0, jnp.int32(-2147483648), jnp.int32(-1))
    return plsc.bitcast(s ^ mask, jnp.float32)


def _sc_topk_body(pre_ref, cvals_ref, cidx_ref, aux_ref,
                  data, gmaxu, cand, candu, hi_idx, eq_idx, ovals, oidx, auxb):
    wid = lax.axis_index("s") * NC + lax.axis_index("c")
    iota = lax.iota(jnp.int32, L)
    ninf = jnp.full((L,), -jnp.inf, jnp.float32)
    zeros_i = jnp.zeros((L,), jnp.int32)

    def count_ge(ref, nvecs, midv):
        def cb(k, acc):
            g = ref[pl.ds(k * L, L)]
            return acc + (g >= midv).astype(jnp.int32)
        accv = lax.fori_loop(0, nvecs, cb, zeros_i)
        return plsc.cumsum(accv)[L - 1]

    def bsearch(ref, nvecs, iters, target):
        def bs(_, lohi):
            lo, hi = lohi
            mid = lo + lax.shift_right_logical(hi - lo, jnp.uint32(1))
            midv = jnp.full((L,), mid, jnp.uint32)
            c = count_ge(ref, nvecs, midv)
            take = c >= target
            return (jnp.where(take, mid, lo), jnp.where(take, hi, mid))
        lo, _ = lax.fori_loop(
            0, iters, bs, (jnp.uint32(0), jnp.uint32(0xFFFFFFFF)))
        return lo

    def append(dst, off_scalar, idxv, msk, cap):
        plsc.store_compressed(dst.at[pl.ds(off_scalar, L)], idxv, mask=msk)
        pc = plsc.all_reduce_population_count(msk)
        return jnp.minimum(off_scalar + pc[0], cap)

    def per_row(j, _):
        row = wid * ROWS_PER_W + j
        pltpu.sync_copy(pre_ref.at[row], data)

        def aux_scan(want_nonneg):
            def cond(c):
                i, off = c
                return (off < AUX_K) & (i < NV)
            def body(c):
                i, off = c
                v = data[pl.ds(i * L, L)]
                bits = plsc.bitcast(v, jnp.int32)
                msk = (bits >= 0) if want_nonneg else (bits < 0)
                posidx = jnp.full((L,), i * L, jnp.int32) + iota
                off = append(auxb, off, posidx, msk, AUX_K)
                return (i + 1, off)
            return body, cond
        b1, c1 = aux_scan(True)
        _, aoff = lax.while_loop(c1, b1, (jnp.int32(0), jnp.int32(0)))
        b2, c2 = aux_scan(False)
        lax.while_loop(c2, b2, (jnp.int32(0), aoff))

        def pa(i, carry):
            base = i * (NACC * L)
            return tuple(
                jnp.maximum(carry[t], data[pl.ds(base + t * L, L)])
                for t in range(NACC))
        maxes = lax.fori_loop(0, NV // NACC, pa, (ninf,) * NACC)
        for t in range(NACC):
            gmaxu[pl.ds(t * L, L)] = _to_u32(maxes[t])

        t1 = bsearch(gmaxu, NACC, 18, MULTI_K)
        thrv = _from_u32(jnp.full((L,), t1, jnp.uint32))

        def pb(g, off):
            base = g * (4 * L)
            vs = [data[pl.ds(base + k * L, L)] for k in range(4)]
            ms = [v >= thrv for v in vs]
            anym = (ms[0] | ms[1]) | (ms[2] | ms[3])
            pca = plsc.all_reduce_population_count(anym)

            def app():
                o = off
                for k in range(4):
                    idxv = jnp.full((L,), base + k * L, jnp.int32) + iota
                    plsc.store_compressed(
                        cand.at[pl.ds(o, L)], idxv, mask=ms[k])
                    pc = plsc.all_reduce_population_count(ms[k])
                    o = jnp.minimum(o + pc[0], CAP)
                return o

            return lax.cond(pca[0] > 0, app, lambda: off)
        count = lax.fori_loop(0, NV // 4, pb, jnp.int32(0), unroll=2)
        countv = jnp.full((L,), count, jnp.int32)

        def emit_main():
            for k in range(CANDS // L):
                lanepos = jnp.full((L,), k * L, jnp.int32) + iota
                valid = lanepos < countv
                idxv = cand[pl.ds(k * L, L)]
                idxs = jnp.where(valid, idxv, 0)
                vals = plsc.load_gather(data, [idxs])
                ovals[pl.ds(k * L, L)] = jnp.where(valid, vals, ninf)
                oidx[pl.ds(k * L, L)] = idxs

        def hi_eq_compact_emit(src_u, src_idx, nvecs, tstar, nv_is_row):
            tsv = jnp.full((L,), tstar, jnp.uint32)
            def cc(k, carry):
                hoff, eoff = carry
                u = src_u(k)
                idxv = src_idx(k)
                mh = u > tsv
                me = u == tsv
                hoff = append(hi_idx, hoff, idxv, mh, MULTI_K + L)
                eoff = append(eq_idx, eoff, idxv, me, MULTI_K)
                return (hoff, eoff)
            hoff, eoff = lax.fori_loop(
                0, nvecs, cc, (jnp.int32(0), jnp.int32(0)))
            hv = jnp.full((L,), hoff, jnp.int32)
            ev = jnp.full((L,), eoff, jnp.int32)
            for k in range(MULTI_K // L):
                lanepos = jnp.full((L,), k * L, jnp.int32) + iota
                validh = lanepos < hv
                valide = lanepos < ev
                ih = jnp.where(validh, hi_idx[pl.ds(k * L, L)], 0)
                ie = jnp.where(valide, eq_idx[pl.ds(k * L, L)], 0)
                if nv_is_row:
                    vh = _from_u32(plsc.bitcast(
                        plsc.load_gather(data, [ih]), jnp.uint32))
                    ve = _from_u32(plsc.bitcast(
                        plsc.load_gather(data, [ie]), jnp.uint32))
                else:
                    vh = plsc.load_gather(data, [ih])
                    ve = plsc.load_gather(data, [ie])
                ovals[pl.ds(k * L, L)] = jnp.where(validh, vh, ninf)
                oidx[pl.ds(k * L, L)] = ih
                ovals[pl.ds(MULTI_K + k * L, L)] = jnp.where(valide, ve, ninf)
                oidx[pl.ds(MULTI_K + k * L, L)] = ie

        def emit_refine():
            nk = lax.shift_right_logical(count + (L - 1), jnp.int32(4))
            def uf(k, _):
                lanepos = jnp.full((L,), 0, jnp.int32) + iota + k * L
                valid = lanepos < countv
                idxv = cand[pl.ds(k * L, L)]
                idxs = jnp.where(valid, idxv, 0)
                vals = plsc.load_gather(data, [idxs])
                u = _to_u32(vals)
                candu[pl.ds(k * L, L)] = jnp.where(valid, u, jnp.uint32(0))
                return 0
            lax.fori_loop(0, nk, uf, 0)
            tstar = bsearch(candu, nk, 32, MULTI_K)
            hi_eq_compact_emit(
                lambda k: candu[pl.ds(k * L, L)],
                lambda k: cand[pl.ds(k * L, L)],
                nk, tstar, False)

        def emit_overflow():
            def ip(i, _):
                v = data[pl.ds(i * L, L)]
                data[pl.ds(i * L, L)] = plsc.bitcast(
                    plsc.bitcast(_to_u32(v), jnp.int32), jnp.float32)
                return 0
            lax.fori_loop(0, NV, ip, 0)
            def src_u(k):
                return plsc.bitcast(data[pl.ds(k * L, L)], jnp.uint32)
            def count_ge_row(midv):
                def cb(k, acc):
                    return acc + (src_u(k) >= midv).astype(jnp.int32)
                return jnp.sum(lax.fori_loop(0, NV, cb, zeros_i))
            def bs(_, lohi):
                lo, hi = lohi
                mid = lo + lax.shift_right_logical(hi - lo, jnp.uint32(1))
                c = count_ge_row(jnp.full((L,), mid, jnp.uint32))
                take = c >= MULTI_K
                return (jnp.where(take, mid, lo), jnp.where(take, hi, mid))
            tstar, _ = lax.fori_loop(
                0, 32, bs, (jnp.uint32(0), jnp.uint32(0xFFFFFFFF)))
            def src_idx(k):
                return jnp.full((L,), k * L, jnp.int32) + iota
            hi_eq_compact_emit(src_u, src_idx, NV, tstar, True)

        lax.cond(
            count <= CANDS, emit_main,
            lambda: lax.cond(count < CAP, emit_refine, emit_overflow))

        pltpu.sync_copy(ovals, cvals_ref.at[row])
        pltpu.sync_copy(oidx, cidx_ref.at[row])
        pltpu.sync_copy(auxb.at[pl.ds(0, AUX_K)], aux_ref.at[row])
        return 0

    lax.fori_loop(0, ROWS_PER_W, per_row, 0)


def _sc_topk(pre_act):
    fn = pl.kernel(
        _sc_topk_body,
        compiler_params=pltpu.CompilerParams(needs_layout_passes=False),
        out_type=[
            jax.ShapeDtypeStruct((B, CANDS), jnp.float32),
            jax.ShapeDtypeStruct((B, CANDS), jnp.int32),
            jax.ShapeDtypeStruct((B, AUX_K), jnp.int32),
        ],
        mesh=plsc.VectorSubcoreMesh(
            core_axis_name="c", subcore_axis_name="s", num_cores=NC),
        scratch_types=[
            pltpu.VMEM((M,), jnp.float32),
            pltpu.VMEM((NACC * L,), jnp.uint32),
            pltpu.VMEM((CAP + L,), jnp.int32),
            pltpu.VMEM((CAP + L,), jnp.uint32),
            pltpu.VMEM((MULTI_K + 2 * L,), jnp.int32),
            pltpu.VMEM((MULTI_K + L,), jnp.int32),
            pltpu.VMEM((CANDS,), jnp.float32),
            pltpu.VMEM((CANDS,), jnp.int32),
            pltpu.VMEM((AUXBUF,), jnp.int32),
        ],
    )
    return fn(pre_act)


SCH = B * K // 128


def _sc_scatter_body(idx_hbm, val_hbm, flat_ref, idxv, valv, sem):
    wid = lax.axis_index("s") * NC + lax.axis_index("c")
    chunks = SCH // NW
    base_c = wid * chunks
    iota = lax.iota(jnp.int32, L)
    pltpu.sync_copy(idx_hbm.at[pl.ds(base_c, chunks)], idxv)
    pltpu.sync_copy(val_hbm.at[pl.ds(base_c, chunks)], valv)
    for c in range(chunks):
        for q in range(128 // L):
            pos = jnp.full((L,), (base_c + c) * 128 + q * L, jnp.int32) + iota
            rowv = lax.shift_right_logical(pos, 6)
            basev = lax.shift_left(rowv, 16)
            idxv[c, pl.ds(q * L, L)] = idxv[c, pl.ds(q * L, L)] + basev
    for c in range(chunks):
        pltpu.async_copy(valv.at[c], flat_ref.at[idxv.at[c]], sem).wait()


def _sc_scatter(idx_pad, val_pad, flat_ref):
    fn = pl.kernel(
        _sc_scatter_body,
        compiler_params=pltpu.CompilerParams(needs_layout_passes=False),
        out_type=(),
        mesh=plsc.VectorSubcoreMesh(
            core_axis_name="c", subcore_axis_name="s", num_cores=NC),
        scratch_types=[
            pltpu.VMEM((SCH // NW, 128), jnp.int32),
            pltpu.VMEM((SCH // NW, 128), jnp.float32),
            pltpu.SemaphoreType.DMA,
        ],
    )
    fn(idx_pad, val_pad, flat_ref)


RCH = 64


def _sc_recon_body(wdt_hbm, idx_hbm, val_hbm, bias_hbm, r_hbm, mr_hbm,
                   idxv, valv, gat0, gat1, bias, rbuf, mrbuf, sem0, sem1):
    wid = lax.axis_index("s") * NC + lax.axis_index("c")
    pltpu.sync_copy(bias_hbm, bias)
    nd = INPUT_DIM // L
    gats = (gat0, gat1)
    sems = (sem0, sem1)

    def per_row(j, _):
        row = wid * ROWS_PER_W + j
        pltpu.sync_copy(idx_hbm.at[row], idxv)
        pltpu.sync_copy(val_hbm.at[row], valv)
        descs = []
        for c in range(2):
            descs.append(pltpu.async_copy(
                wdt_hbm.at[idxv.at[pl.ds(c * RCH, RCH)]], gats[c % 2],
                sems[c % 2]))
        accs = tuple(bias[pl.ds(d * L, L)] for d in range(nd))
        for c in range(MULTI_K // RCH):
            descs[c].wait()

            def macbody(t, accs, _c=c):
                sv = plsc.load_gather(
                    valv, [jnp.full((L,), _c * RCH + t, jnp.int32)])
                return tuple(
                    a + sv * gats[_c % 2][t, pl.ds(d * L, L)]
                    for d, a in enumerate(accs))

            accs = lax.fori_loop(0, RCH, macbody, accs, unroll=4)
            if c + 2 < MULTI_K // RCH:
                descs.append(pltpu.async_copy(
                    wdt_hbm.at[idxv.at[pl.ds((c + 2) * RCH, RCH)]],
                    gats[c % 2], sems[c % 2]))
            if c == K // RCH - 1:
                for d in range(nd):
                    rbuf[pl.ds(d * L, L)] = accs[d]
        for d in range(nd):
            mrbuf[pl.ds(d * L, L)] = accs[d]
        pltpu.sync_copy(rbuf, r_hbm.at[row])
        pltpu.sync_copy(mrbuf, mr_hbm.at[row])
        return 0

    lax.fori_loop(0, ROWS_PER_W, per_row, 0)


def _sc_recon(W_dec_T, mk_idx, mk_vals_relu, input_bias):
    fn = pl.kernel(
        _sc_recon_body,
        compiler_params=pltpu.CompilerParams(needs_layout_passes=False),
        out_type=[
            jax.ShapeDtypeStruct((B, INPUT_DIM), jnp.float32),
            jax.ShapeDtypeStruct((B, INPUT_DIM), jnp.float32),
        ],
        mesh=plsc.VectorSubcoreMesh(
            core_axis_name="c", subcore_axis_name="s", num_cores=NC),
        scratch_types=[
            pltpu.VMEM((MULTI_K,), jnp.int32),
            pltpu.VMEM((MULTI_K,), jnp.float32),
            pltpu.VMEM((RCH, INPUT_DIM), jnp.float32),
            pltpu.VMEM((RCH, INPUT_DIM), jnp.float32),
            pltpu.VMEM((INPUT_DIM,), jnp.float32),
            pltpu.VMEM((INPUT_DIM,), jnp.float32),
            pltpu.VMEM((INPUT_DIM,), jnp.float32),
            pltpu.SemaphoreType.DMA,
            pltpu.SemaphoreType.DMA,
        ],
    )
    return fn(W_dec_T, mk_idx, mk_vals_relu, input_bias)


def kernel(x, W_enc, W_dec, input_bias, neuron_bias):
    pre_act, act_zeros, W_dec_T = _encoder_matmul(
        x, W_enc, W_dec, input_bias, neuron_bias)

    cvals, cidx, aux_indices = _sc_topk(pre_act)

    mk_vals, mk_pos = jax.lax.top_k(cvals, MULTI_K)
    mk_idx = jnp.take_along_axis(cidx, mk_pos, axis=1)
    mk_vals_relu = jax.nn.relu(mk_vals)
    topk_indices = mk_idx[:, :K]
    topk_values = mk_vals_relu[:, :K]

    flat_ref = jax.new_ref(act_zeros.reshape(B * M))
    _sc_scatter(topk_indices.reshape(SCH, 128),
                topk_values.reshape(SCH, 128), flat_ref)
    activations = flat_ref[...].reshape(B, M)

    reconstruction, multik_reconstruction = _sc_recon(
        W_dec_T, mk_idx, mk_vals_relu, input_bias)

    aux_values = jnp.zeros((B, AUX_K), dtype=jnp.float32)
    return (reconstruction, activations, topk_indices, topk_values,
            multik_reconstruction, aux_indices, aux_values)
